# Initial kernel scaffold; baseline (speedup 1.0000x reference)
import jax
import jax.numpy as jnp
from jax.experimental import pallas as pl


def kernel(idx, table):
    raise NotImplementedError("write your pallas kernel here")


# speedup vs baseline: 4.9692x; 4.9692x over previous
import functools

import jax
import jax.numpy as jnp
from jax import lax
from jax.experimental import pallas as pl
from jax.experimental.pallas import tpu as pltpu
from jax.experimental.pallas import tpu_sc as plsc

VOCAB = 100000
N_EMBD = 64
BATCH = 16384
SEQ = 200

NC = 2
NS = 16
NW = NC * NS

B_TOTAL = BATCH * SEQ
B_PER_W = B_TOTAL // NW
K = 8
CHUNK = K * 128
N_CHUNKS = B_PER_W // CHUNK


def _emb_kernel(idx_hbm, table_hbm, out_hbm, idx_v, rows_v, sem):
  wid = lax.axis_index("s") * NC + lax.axis_index("c")
  base = wid * B_PER_W

  def body(g, _):
    row0 = base + g * CHUNK
    chunk_row = pl.multiple_of(row0 // 128, 8)
    pltpu.sync_copy(idx_hbm.at[pl.ds(chunk_row, K)], idx_v)
    copies = [
        pltpu.async_copy(
            table_hbm.at[idx_v.at[j]],
            rows_v.at[pl.ds(j * 128, 128)],
            sem,
        )
        for j in range(K)
    ]
    for c in copies:
      c.wait()
    pltpu.sync_copy(rows_v, out_hbm.at[pl.ds(row0, CHUNK)])
    return ()

  lax.fori_loop(0, N_CHUNKS, body, ())


@jax.jit
def _embedding_lookup(idx_flat, table):
  mesh = plsc.VectorSubcoreMesh(
      core_axis_name="c", subcore_axis_name="s", num_cores=NC, num_subcores=NS
  )
  f = pl.kernel(
      _emb_kernel,
      out_type=jax.ShapeDtypeStruct((B_TOTAL, N_EMBD), jnp.float32),
      mesh=mesh,
      scratch_types=[
          pltpu.VMEM((K, 128), jnp.int32),
          pltpu.VMEM((CHUNK, N_EMBD), jnp.float32),
          pltpu.SemaphoreType.DMA,
      ],
      compiler_params=pltpu.CompilerParams(use_tc_tiling_on_sc=False),
  )
  return f(idx_flat, table)


def kernel(idx, table):
  out = _embedding_lookup(idx.reshape(B_TOTAL // 128, 128), table)
  return out.reshape(BATCH, SEQ, N_EMBD)


# speedup vs baseline: 5.1326x; 1.0329x over previous
import functools

import jax
import jax.numpy as jnp
from jax import lax
from jax.experimental import pallas as pl
from jax.experimental.pallas import tpu as pltpu
from jax.experimental.pallas import tpu_sc as plsc

VOCAB = 100000
N_EMBD = 64
BATCH = 16384
SEQ = 200

NC = 2
NS = 16
NW = NC * NS

B_TOTAL = BATCH * SEQ
B_PER_W = B_TOTAL // NW
KH = 4
HC = KH * 128
N_OUTER = B_PER_W // (2 * HC)


def _emb_kernel(idx_hbm, table_hbm, out_hbm,
                idx0, idx1, rows0, rows1,
                isem0, isem1, gsem0, gsem1, wsem0, wsem1):
  wid = lax.axis_index("s") * NC + lax.axis_index("c")
  base = wid * B_PER_W
  cbase = base // 128

  idx_bufs = (idx0, idx1)
  rows_bufs = (rows0, rows1)
  isems = (isem0, isem1)
  gsems = (gsem0, gsem1)
  wsems = (wsem0, wsem1)

  def idx_slice(t):
    return idx_hbm.at[pl.ds(pl.multiple_of(cbase + t * KH, 4), KH)]

  for b in range(2):
    pltpu.async_copy(idx_slice(b), idx_bufs[b], isems[b])

  def body(i, _):
    t0 = 2 * i
    gathers = []
    for b in range(2):
      @pl.when(i > 0)
      def _wb_done():
        pltpu.make_async_copy(
            rows_bufs[b], out_hbm.at[pl.ds(0, HC)], wsems[b]
        ).wait()
      pltpu.make_async_copy(idx_slice(0), idx_bufs[b], isems[b]).wait()
      for j in range(KH):
        gathers.append(pltpu.async_copy(
            table_hbm.at[idx_bufs[b].at[j]],
            rows_bufs[b].at[pl.ds(j * 128, 128)],
            gsems[b],
        ))
    for b in range(2):
      for j in range(KH):
        gathers[b * KH + j].wait()
      row0 = base + (t0 + b) * HC
      pltpu.async_copy(rows_bufs[b], out_hbm.at[pl.ds(row0, HC)], wsems[b])
      @pl.when(i < N_OUTER - 1)
      def _prefetch():
        pltpu.async_copy(idx_slice(t0 + 2 + b), idx_bufs[b], isems[b])
    return ()

  lax.fori_loop(0, N_OUTER, body, ())

  for b in range(2):
    pltpu.make_async_copy(
        rows_bufs[b], out_hbm.at[pl.ds(0, HC)], wsems[b]
    ).wait()


@jax.jit
def _embedding_lookup(idx_flat, table):
  mesh = plsc.VectorSubcoreMesh(
      core_axis_name="c", subcore_axis_name="s", num_cores=NC, num_subcores=NS
  )
  f = pl.kernel(
      _emb_kernel,
      out_type=jax.ShapeDtypeStruct((B_TOTAL, N_EMBD), jnp.float32),
      mesh=mesh,
      scratch_types=[
          pltpu.VMEM((KH, 128), jnp.int32),
          pltpu.VMEM((KH, 128), jnp.int32),
          pltpu.VMEM((HC, N_EMBD), jnp.float32),
          pltpu.VMEM((HC, N_EMBD), jnp.float32),
          pltpu.SemaphoreType.DMA,
          pltpu.SemaphoreType.DMA,
          pltpu.SemaphoreType.DMA,
          pltpu.SemaphoreType.DMA,
          pltpu.SemaphoreType.DMA,
          pltpu.SemaphoreType.DMA,
      ],
      compiler_params=pltpu.CompilerParams(use_tc_tiling_on_sc=False),
  )
  return f(idx_flat, table)


def kernel(idx, table):
  out = _embedding_lookup(idx.reshape(B_TOTAL // 128, 128), table)
  return out.reshape(BATCH, SEQ, N_EMBD)


# speedup vs baseline: 5.1438x; 1.0022x over previous
import functools

import jax
import jax.numpy as jnp
from jax import lax
from jax.experimental import pallas as pl
from jax.experimental.pallas import tpu as pltpu
from jax.experimental.pallas import tpu_sc as plsc

VOCAB = 100000
N_EMBD = 64
BATCH = 16384
SEQ = 200

NC = 2
NS = 16
NW = NC * NS

B_TOTAL = BATCH * SEQ
B_PER_W = B_TOTAL // NW
HC = 512
N_OUTER = B_PER_W // (2 * HC)


def _emb_kernel(idx_hbm, table_hbm, out_hbm,
                idx0, idx1, rows0, rows1,
                isem0, isem1, gsem0, gsem1, wsem0, wsem1):
  wid = lax.axis_index("s") * NC + lax.axis_index("c")
  base = wid * B_PER_W

  idx_bufs = (idx0, idx1)
  rows_bufs = (rows0, rows1)
  isems = (isem0, isem1)
  gsems = (gsem0, gsem1)
  wsems = (wsem0, wsem1)

  def idx_slice(t):
    return idx_hbm.at[pl.ds(pl.multiple_of(base + t * HC, 8), HC)]

  for b in range(2):
    pltpu.async_copy(idx_slice(b), idx_bufs[b], isems[b])

  def body(i, _):
    t0 = 2 * i
    gathers = []
    for b in range(2):
      @pl.when(i > 0)
      def _wb_done():
        pltpu.make_async_copy(
            rows_bufs[b], out_hbm.at[pl.ds(0, HC)], wsems[b]
        ).wait()
      pltpu.make_async_copy(idx_slice(0), idx_bufs[b], isems[b]).wait()
      gathers.append(pltpu.async_copy(
          table_hbm.at[idx_bufs[b]], rows_bufs[b], gsems[b]))
    for b in range(2):
      gathers[b].wait()
      row0 = base + (t0 + b) * HC
      pltpu.async_copy(rows_bufs[b], out_hbm.at[pl.ds(row0, HC)], wsems[b])
      @pl.when(i < N_OUTER - 1)
      def _prefetch():
        pltpu.async_copy(idx_slice(t0 + 2 + b), idx_bufs[b], isems[b])
    return ()

  lax.fori_loop(0, N_OUTER, body, ())

  for b in range(2):
    pltpu.make_async_copy(
        rows_bufs[b], out_hbm.at[pl.ds(0, HC)], wsems[b]
    ).wait()


@jax.jit
def _embedding_lookup(idx_flat, table):
  mesh = plsc.VectorSubcoreMesh(
      core_axis_name="c", subcore_axis_name="s", num_cores=NC, num_subcores=NS
  )
  f = pl.kernel(
      _emb_kernel,
      out_type=jax.ShapeDtypeStruct((B_TOTAL, N_EMBD), jnp.float32),
      mesh=mesh,
      scratch_types=[
          pltpu.VMEM((HC,), jnp.int32),
          pltpu.VMEM((HC,), jnp.int32),
          pltpu.VMEM((HC, N_EMBD), jnp.float32),
          pltpu.VMEM((HC, N_EMBD), jnp.float32),
          pltpu.SemaphoreType.DMA,
          pltpu.SemaphoreType.DMA,
          pltpu.SemaphoreType.DMA,
          pltpu.SemaphoreType.DMA,
          pltpu.SemaphoreType.DMA,
          pltpu.SemaphoreType.DMA,
      ],
      compiler_params=pltpu.CompilerParams(use_tc_tiling_on_sc=False),
  )
  return f(idx_flat, table)


def kernel(idx, table):
  out = _embedding_lookup(idx.reshape(-1), table)
  return out.reshape(BATCH, SEQ, N_EMBD)
